# Initial kernel scaffold; baseline (speedup 1.0000x reference)
import jax
import jax.numpy as jnp
from jax.experimental import pallas as pl


def kernel(x, edge_index, W1, b1, W2, b2):
    raise NotImplementedError("write your pallas kernel here")


# speedup vs baseline: 3.7448x; 3.7448x over previous
import functools

import jax
import jax.numpy as jnp
from jax import lax
from jax.experimental import pallas as pl
from jax.experimental.pallas import tpu as pltpu
from jax.experimental.pallas import tpu_sc as plsc

N = 10000
D = 128
C = 16
E = 320000

NC = 2
NS = 16
NW = NC * NS

BLK = 128
BPT_AGG = -(-(-(-E // (NW * BLK))) // 8) * 8
E_PAD = NW * BPT_AGG * BLK
EROWS = E_PAD // BLK
BPT_DEG = EROWS // NS

N_PAD = 10240
NPT = N_PAD // NS


def _deg_body(src_hbm, dst_hbm, out_hbm, sidx_v, didx_v, degs_v, degd_v,
              tmp_v, acc_v, deg_sh):
    c = lax.axis_index("c")
    s = lax.axis_index("s")
    zeros = jnp.zeros((16,), jnp.float32)
    ones = jnp.full((16,), 1.0, jnp.float32)
    for k in range(N_PAD // 16):
        degs_v[pl.ds(k * 16, 16)] = zeros
        degd_v[pl.ds(k * 16, 16)] = zeros
    pltpu.sync_copy(src_hbm.at[pl.ds(s * BPT_DEG, BPT_DEG)], sidx_v)
    pltpu.sync_copy(dst_hbm.at[pl.ds(s * BPT_DEG, BPT_DEG)], didx_v)

    def step(b, carry):
        for j in range(BLK // 16):
            sv = sidx_v[b, pl.ds(j * 16, 16)]
            plsc.addupdate_scatter(degs_v, [sv], ones)
            dv = didx_v[b, pl.ds(j * 16, 16)]
            plsc.addupdate_scatter(degd_v, [dv], ones)
        return carry

    lax.fori_loop(0, BPT_DEG, step, 0)

    pltpu.sync_copy(degs_v, deg_sh.at[0, s])
    pltpu.sync_copy(degd_v, deg_sh.at[1, s])
    plsc.subcore_barrier()
    for a in range(2):
        pltpu.sync_copy(deg_sh.at[a, 0, pl.ds(s * NPT, NPT)], acc_v.at[a])
        for t in range(1, NS):
            pltpu.sync_copy(deg_sh.at[a, t, pl.ds(s * NPT, NPT)], tmp_v)
            for k in range(NPT // 16):
                sl = pl.ds(k * 16, 16)
                acc_v[a, sl] = acc_v[a, sl] + tmp_v[sl]
    pltpu.sync_copy(acc_v.at[c], out_hbm.at[c, pl.ds(s * NPT, NPT)])


_deg_call = functools.partial(
    pl.kernel,
    out_type=jax.ShapeDtypeStruct((NC, N_PAD), jnp.float32),
    mesh=plsc.VectorSubcoreMesh(core_axis_name="c", subcore_axis_name="s"),
    scratch_types=[
        pltpu.VMEM((BPT_DEG, BLK), jnp.int32),
        pltpu.VMEM((BPT_DEG, BLK), jnp.int32),
        pltpu.VMEM((N_PAD,), jnp.float32),
        pltpu.VMEM((N_PAD,), jnp.float32),
        pltpu.VMEM((NPT,), jnp.float32),
        pltpu.VMEM((2, NPT), jnp.float32),
        pltpu.VMEM_SHARED((2, NS, N_PAD), jnp.float32),
    ],
    compiler_params=pltpu.CompilerParams(needs_layout_passes=False),
)(_deg_body)


def _agg_body(h_hbm, src_hbm, dst_hbm, out_hbm, sidx_v, didx_v, rows_v,
              zero_v, agg_sh, sem):
    c = lax.axis_index("c")
    s = lax.axis_index("s")
    wid = c * NS + s
    for i in range(16):
        for j in range(D // 16):
            zero_v[i, pl.ds(j * 16, 16)] = jnp.zeros((16,), jnp.float32)
    for k in range(NPT // 16):
        pltpu.sync_copy(zero_v, agg_sh.at[pl.ds(s * NPT + k * 16, 16)])
    plsc.subcore_barrier()

    r0 = wid * BPT_AGG
    pltpu.sync_copy(src_hbm.at[pl.ds(r0, BPT_AGG)], sidx_v)
    pltpu.sync_copy(dst_hbm.at[pl.ds(r0, BPT_AGG)], didx_v)

    def step(b, carry):
        pltpu.async_copy(h_hbm.at[sidx_v.at[b]], rows_v, sem).wait()
        pltpu.sync_copy(rows_v, agg_sh.at[didx_v.at[b]], add=True)
        return carry

    lax.fori_loop(0, BPT_AGG, step, 0)
    plsc.subcore_barrier()
    pltpu.sync_copy(agg_sh.at[pl.ds(s * NPT, NPT)],
                    out_hbm.at[c, pl.ds(s * NPT, NPT)])


_agg_call = functools.partial(
    pl.kernel,
    out_type=jax.ShapeDtypeStruct((NC, N_PAD, D), jnp.float32),
    mesh=plsc.VectorSubcoreMesh(core_axis_name="c", subcore_axis_name="s"),
    scratch_types=[
        pltpu.VMEM((BPT_AGG, BLK), jnp.int32),
        pltpu.VMEM((BPT_AGG, BLK), jnp.int32),
        pltpu.VMEM((BLK, D), jnp.float32),
        pltpu.VMEM((16, D), jnp.float32),
        pltpu.VMEM_SHARED((N_PAD, D), jnp.float32),
        pltpu.SemaphoreType.DMA,
    ],
)(_agg_body)


def _norm(deg_col):
    return jnp.where(deg_col > 0.0,
                     lax.rsqrt(jnp.maximum(deg_col, 1.0)), 0.0)


def _scale_body(x_ref, deg_ref, o_ref):
    o_ref[...] = x_ref[...] * _norm(deg_ref[...])


def _dense_body(agg_ref, deg_ref, w1_ref, b1_ref, w2_ref, b2_ref, o_ref):
    a = agg_ref[0] + agg_ref[1]
    t = a * _norm(deg_ref[...])
    y = jnp.dot(t, w1_ref[...], preferred_element_type=jnp.float32)
    y = jnp.maximum(y + b1_ref[...], 0.0)
    rows = lax.broadcasted_iota(jnp.int32, (N_PAD, 1), 0)
    y = jnp.where(rows < N, y, 0.0)
    m = jnp.sum(y, axis=0, keepdims=True) * (1.0 / N)
    o_ref[...] = jnp.dot(m, w2_ref[...], preferred_element_type=jnp.float32) \
        + b2_ref[...]


def kernel(x, edge_index, W1, b1, W2, b2):
    src = edge_index[0]
    dst = edge_index[1]
    pad = E_PAD - E
    padv = jnp.full((pad,), N, jnp.int32)
    src_p = jnp.concatenate([src, padv]).reshape(EROWS, BLK)
    dst_p = jnp.concatenate([dst, padv]).reshape(EROWS, BLK)
    x_p = jnp.zeros((N_PAD, D), jnp.float32).at[:N].set(x)

    deg = _deg_call(src_p, dst_p)

    h = pl.pallas_call(
        _scale_body,
        out_shape=jax.ShapeDtypeStruct((N_PAD, D), jnp.float32),
    )(x_p, deg[0].reshape(N_PAD, 1))

    agg = _agg_call(h, src_p, dst_p)

    out = pl.pallas_call(
        _dense_body,
        out_shape=jax.ShapeDtypeStruct((1, C), jnp.float32),
    )(agg, deg[1].reshape(N_PAD, 1), W1, b1.reshape(1, D), W2,
      b2.reshape(1, C))
    return out


# speedup vs baseline: 3.9448x; 1.0534x over previous
import functools

import jax
import jax.numpy as jnp
from jax import lax
from jax.experimental import pallas as pl
from jax.experimental.pallas import tpu as pltpu
from jax.experimental.pallas import tpu_sc as plsc

N = 10000
D = 128
C = 16
E = 320000

NC = 2
NS = 16
NW = NC * NS

BLK = 128
BPT_AGG = -(-(-(-E // (NW * BLK))) // 8) * 8
E_PAD = NW * BPT_AGG * BLK
EROWS = E_PAD // BLK
BPT_DEG = EROWS // NS

N_PAD = 10240
NPT = N_PAD // NS


def _deg_body(src_hbm, dst_hbm, out_hbm, sidx_v, didx_v, deg_v):
    c = lax.axis_index("c")
    s = lax.axis_index("s")
    zeros = jnp.zeros((16,), jnp.float32)
    ones = jnp.full((16,), 1.0, jnp.float32)
    for k in range(2 * N_PAD // 16):
        deg_v[pl.ds(k * 16, 16)] = zeros
    pltpu.sync_copy(src_hbm.at[pl.ds(s * BPT_DEG, BPT_DEG)], sidx_v)
    pltpu.sync_copy(dst_hbm.at[pl.ds(s * BPT_DEG, BPT_DEG)], didx_v)

    def step(b, carry):
        for j in range(BLK // 16):
            sv = sidx_v[b, pl.ds(j * 16, 16)]
            plsc.addupdate_scatter(deg_v, [sv], ones)
            dv = didx_v[b, pl.ds(j * 16, 16)]
            plsc.addupdate_scatter(deg_v, [dv + N_PAD], ones)
        return carry

    lax.fori_loop(0, BPT_DEG, step, 0)
    pltpu.sync_copy(deg_v.at[pl.ds(c * N_PAD, N_PAD)], out_hbm.at[c, s])


_deg_call = functools.partial(
    pl.kernel,
    out_type=jax.ShapeDtypeStruct((NC, NS, N_PAD), jnp.float32),
    mesh=plsc.VectorSubcoreMesh(core_axis_name="c", subcore_axis_name="s"),
    scratch_types=[
        pltpu.VMEM((BPT_DEG, BLK), jnp.int32),
        pltpu.VMEM((BPT_DEG, BLK), jnp.int32),
        pltpu.VMEM((2 * N_PAD,), jnp.float32),
    ],
    compiler_params=pltpu.CompilerParams(needs_layout_passes=False),
)(_deg_body)


K_RING = 2
NPHASE = 2
HBPT = BPT_AGG // NPHASE
NG_AGG = HBPT // K_RING


def _agg_body(h_hbm, src_hbm, dst_hbm, out_hbm, sidx_v, didx_v, rows_v,
              zero_v, agg_sh, gsem):
    c = lax.axis_index("c")
    s = lax.axis_index("s")
    wid = c * NS + s
    for i in range(16):
        for j in range(D // 16):
            zero_v[i, pl.ds(j * 16, 16)] = jnp.zeros((16,), jnp.float32)
    for k in range(NPT // 16):
        pltpu.sync_copy(zero_v, agg_sh.at[pl.ds(s * NPT + k * 16, 16)])
    plsc.subcore_barrier()

    r0 = wid * BPT_AGG
    for ph in range(NPHASE):
        pltpu.sync_copy(src_hbm.at[pl.ds(r0 + ph * HBPT, HBPT)], sidx_v)
        pltpu.sync_copy(dst_hbm.at[pl.ds(r0 + ph * HBPT, HBPT)], didx_v)

        for k in range(K_RING):
            pltpu.async_copy(h_hbm.at[sidx_v.at[k]], rows_v.at[k],
                             gsem.at[k])

        def group(g, carry):
            base = g * K_RING
            for k in range(K_RING):
                i = base + k
                pltpu.make_async_copy(h_hbm.at[sidx_v.at[i]], rows_v.at[k],
                                      gsem.at[k]).wait()
                pltpu.sync_copy(rows_v.at[k], agg_sh.at[didx_v.at[i]],
                                add=True)

                @pl.when(g < NG_AGG - 1)
                def _():
                    pltpu.async_copy(h_hbm.at[sidx_v.at[i + K_RING]],
                                     rows_v.at[k], gsem.at[k])
            return carry

        lax.fori_loop(0, NG_AGG, group, 0)
    plsc.subcore_barrier()
    pltpu.sync_copy(agg_sh.at[pl.ds(s * NPT, NPT)],
                    out_hbm.at[c, pl.ds(s * NPT, NPT)])


_agg_call = functools.partial(
    pl.kernel,
    out_type=jax.ShapeDtypeStruct((NC, N_PAD, D), jnp.float32),
    mesh=plsc.VectorSubcoreMesh(core_axis_name="c", subcore_axis_name="s"),
    scratch_types=[
        pltpu.VMEM((HBPT, BLK), jnp.int32),
        pltpu.VMEM((HBPT, BLK), jnp.int32),
        pltpu.VMEM((K_RING, BLK, D), jnp.float32),
        pltpu.VMEM((16, D), jnp.float32),
        pltpu.VMEM_SHARED((N_PAD, D), jnp.float32),
        pltpu.SemaphoreType.DMA((K_RING,)),
    ],
)(_agg_body)


def _norm(deg_col):
    return jnp.where(deg_col > 0.0,
                     lax.rsqrt(jnp.maximum(deg_col, 1.0)), 0.0)


def _col_sum(page):
    ones = jnp.ones((NS, 1), jnp.float32)
    return lax.dot_general(page, ones, (((0,), (0,)), ((), ())),
                           preferred_element_type=jnp.float32)


def _scale_body(x_ref, deg_ref, o_ref, nd_ref):
    od = _col_sum(deg_ref[0])
    idg = _col_sum(deg_ref[1])
    o_ref[...] = x_ref[...] * _norm(od)
    nd_ref[...] = _norm(idg)


def _dense_body(agg_ref, nd_ref, w1_ref, b1_ref, w2_ref, b2_ref, o_ref):
    a = agg_ref[0] + agg_ref[1]
    t = a * nd_ref[...]
    y = jnp.dot(t, w1_ref[...], preferred_element_type=jnp.float32)
    y = jnp.maximum(y + b1_ref[...], 0.0)
    rows = lax.broadcasted_iota(jnp.int32, (N_PAD, 1), 0)
    y = jnp.where(rows < N, y, 0.0)
    m = jnp.sum(y, axis=0, keepdims=True) * (1.0 / N)
    o_ref[...] = jnp.dot(m, w2_ref[...], preferred_element_type=jnp.float32) \
        + b2_ref[...]


def kernel(x, edge_index, W1, b1, W2, b2):
    src = edge_index[0]
    dst = edge_index[1]
    pad = E_PAD - E
    padv = jnp.full((pad,), N, jnp.int32)
    src_p = jnp.concatenate([src, padv]).reshape(EROWS, BLK)
    dst_p = jnp.concatenate([dst, padv]).reshape(EROWS, BLK)
    x_p = jnp.zeros((N_PAD, D), jnp.float32).at[:N].set(x)

    deg = _deg_call(src_p, dst_p)

    h, norm_dst = pl.pallas_call(
        _scale_body,
        out_shape=(jax.ShapeDtypeStruct((N_PAD, D), jnp.float32),
                   jax.ShapeDtypeStruct((N_PAD, 1), jnp.float32)),
    )(x_p, deg)

    agg = _agg_call(h, src_p, dst_p)

    out = pl.pallas_call(
        _dense_body,
        out_shape=jax.ShapeDtypeStruct((1, C), jnp.float32),
    )(agg, norm_dst, W1, b1.reshape(1, D), W2, b2.reshape(1, C))
    return out
